# baseline (speedup 1.0000x reference)
import functools

import jax
import jax.numpy as jnp
from jax import lax
from jax.experimental import pallas as pl
from jax.experimental.pallas import tpu as pltpu
from jax.experimental.pallas import tpu_sc as plsc

_BATCH = 4096
_FIELDS = 26
_EMBED = 32
_N = 1000000
_TOTAL = _BATCH * _FIELDS

_BK = 32768
_NBLK = (_N + _BK - 1) // _BK
_PACKED_ROWS = _NBLK * (_BK // 4)

_Q = _BK // 4
_SH_BLK = _BK.bit_length() - 1
_SH_G = _Q.bit_length() - 1

_NW = 32
_PER_W = _TOTAL // _NW
_CHUNKS = _PER_W // 128


def _t_body(x_ref, o_ref):
    eye = jnp.eye(32, dtype=jnp.float32)
    xT = lax.dot_general(
        x_ref[...], eye, (((0,), (0,)), ((), ())),
        preferred_element_type=jnp.float32)
    q = _BK // 4
    for g in range(4):
        o_ref[:, g * 32:(g + 1) * 32] = xT[g * q:(g + 1) * q, :]


def _tc_pack(wT):
    return pl.pallas_call(
        _t_body,
        out_shape=jax.ShapeDtypeStruct((_PACKED_ROWS, 128), jnp.float32),
        grid=(_NBLK,),
        in_specs=[pl.BlockSpec((32, _BK), lambda j: (0, j))],
        out_specs=pl.BlockSpec((_BK // 4, 128), lambda j: (j, 0)),
    )(wT)


def _bcast(vec, k):
    return lax.gather(
        vec, jnp.full((16, 1), k, jnp.int32),
        lax.GatherDimensionNumbers(
            offset_dims=(), collapsed_slice_dims=(0,), start_index_map=(0,)),
        slice_sizes=(1,),
        mode=lax.GatherScatterMode.PROMISE_IN_BOUNDS)


def _g_body(idxf, w4, out, idxv, rivv, rows_v, opbuf, sem):
    w = lax.axis_index("s") * 2 + lax.axis_index("c")
    base = pl.multiple_of(w * _PER_W, 8)
    pltpu.sync_copy(idxf.at[pl.ds(base, _PER_W)], idxv)
    for m in range(_PER_W // 16):
        v = idxv[pl.ds(m * 16, 16)]
        rivv[pl.ds(m * 16, 16)] = ((v >> _SH_BLK) << _SH_G) + (v & (_Q - 1))

    def chunk(j, carry):
        c0 = pl.multiple_of(j * 128, 8)
        pltpu.async_copy(w4.at[rivv.at[pl.ds(c0, 128)]], rows_v, sem).wait()
        for mm in range(8):
            g16 = (idxv[pl.ds(c0 + mm * 16, 16)] >> _SH_G) & 3
            oh = jnp.int32(1) << g16
            mf = [((oh >> g) & 1).astype(jnp.float32) for g in range(4)]
            for k in range(16):
                l = mm * 16 + k
                mb = [_bcast(mf[g], k) for g in range(4)]
                dst = (l % 4) * 32
                for h in (0, 16):
                    acc = mb[0] * rows_v[l, pl.ds(h, 16)]
                    acc = acc + mb[1] * rows_v[l, pl.ds(32 + h, 16)]
                    acc = acc + mb[2] * rows_v[l, pl.ds(64 + h, 16)]
                    acc = acc + mb[3] * rows_v[l, pl.ds(96 + h, 16)]
                    opbuf[l // 4, pl.ds(dst + h, 16)] = acc
        o0 = pl.multiple_of(w * (_PER_W // 4) + j * 32, 8)
        pltpu.sync_copy(opbuf, out.at[pl.ds(o0, 32), :])
        return carry

    lax.fori_loop(0, _CHUNKS, chunk, 0)


@jax.jit
def _run(idxf, wT):
    w4 = _tc_pack(wT)
    mesh = plsc.VectorSubcoreMesh(core_axis_name="c", subcore_axis_name="s")
    gather = pl.kernel(
        _g_body,
        out_type=jax.ShapeDtypeStruct((_TOTAL // 4, 128), jnp.float32),
        mesh=mesh,
        compiler_params=pltpu.CompilerParams(use_tc_tiling_on_sc=True),
        scratch_types=[
            pltpu.VMEM((_PER_W,), jnp.int32),
            pltpu.VMEM((_PER_W,), jnp.int32),
            pltpu.VMEM((128, 128), jnp.float32),
            pltpu.VMEM((32, 128), jnp.float32),
            pltpu.SemaphoreType.DMA,
        ],
    )
    return gather(idxf, w4)


def kernel(input_tensor, weight):
    idx = input_tensor
    if idx.ndim == 1:
        idx = idx[None, :]
    opack = _run(idx.reshape(-1), weight.T)
    return opack.reshape(*idx.shape, _EMBED)


# speedup vs baseline: 2.1804x; 2.1804x over previous
import functools

import jax
import jax.numpy as jnp
from jax import lax
from jax.experimental import pallas as pl
from jax.experimental.pallas import tpu as pltpu
from jax.experimental.pallas import tpu_sc as plsc

_BATCH = 4096
_FIELDS = 26
_EMBED = 32
_N = 1000000
_TOTAL = _BATCH * _FIELDS

_BK = 32768
_NBLK = (_N + _BK - 1) // _BK
_Q = _BK // 4
_PACKED_ROWS = _NBLK * _Q
_SH_BLK = _BK.bit_length() - 1
_SH_G = _Q.bit_length() - 1

_NW = 32
_PER_W = _TOTAL // _NW
_CHUNKS = _PER_W // 128


def _t_body(x_ref, o_ref):
    x = x_ref[...]
    x4 = jnp.concatenate([x[:, g * _Q:(g + 1) * _Q] for g in range(4)], axis=0)
    o_ref[...] = x4.T


def _tc_pack(wT):
    return pl.pallas_call(
        _t_body,
        out_shape=jax.ShapeDtypeStruct((_PACKED_ROWS, 128), jnp.float32),
        grid=(_NBLK,),
        in_specs=[pl.BlockSpec((32, _BK), lambda j: (0, j))],
        out_specs=pl.BlockSpec((_Q, 128), lambda j: (j, 0)),
    )(wT)


def _g_body(idxf, w4, big, idxv, rivv, rows_v, sem):
    w = lax.axis_index("s") * 2 + lax.axis_index("c")
    base = pl.multiple_of(w * _PER_W, 8)
    pltpu.sync_copy(idxf.at[pl.ds(base, _PER_W)], idxv)
    for m in range(_PER_W // 16):
        v = idxv[pl.ds(m * 16, 16)]
        rivv[pl.ds(m * 16, 16)] = ((v >> _SH_BLK) << _SH_G) + (v & (_Q - 1))

    def fire(j):
        c = pl.multiple_of(j * 128, 8)
        return pltpu.async_copy(
            w4.at[rivv.at[pl.ds(c, 128)]], rows_v.at[j % 2], sem)

    fire(0)

    def chunk(j, carry):
        c0 = pl.multiple_of(j * 128, 8)

        @pl.when(j + 1 < _CHUNKS)
        def _():
            fire(j + 1)

        pltpu.make_async_copy(
            w4.at[rivv.at[pl.ds(c0, 128)]], rows_v.at[j % 2], sem).wait()
        o0 = pl.multiple_of(base + j * 128, 8)
        pltpu.sync_copy(rows_v.at[j % 2], big.at[pl.ds(o0, 128), :])
        return carry

    lax.fori_loop(0, _CHUNKS, chunk, 0)


def _x_body(idx_ref, rows_ref, o_ref):
    f = pl.program_id(0)
    gv = (idx_ref[pl.ds(f, 1), :] >> _SH_G) & 3
    rT = rows_ref[...].T
    mf = [(gv == g).astype(jnp.float32) for g in range(4)]
    mf8 = [jnp.broadcast_to(m, (8, _BATCH)) for m in mf]
    for jg in range(4):
        acc = mf8[0] * rT[jg * 8:(jg + 1) * 8, :]
        for g in range(1, 4):
            acc = acc + mf8[g] * rT[g * 32 + jg * 8:g * 32 + (jg + 1) * 8, :]
        o_ref[0, jg * 8:(jg + 1) * 8, :] = acc


def _tc_extract(idxT, big):
    return pl.pallas_call(
        _x_body,
        out_shape=jax.ShapeDtypeStruct((_FIELDS, _EMBED, _BATCH), jnp.float32),
        grid=(_FIELDS,),
        in_specs=[
            pl.BlockSpec((_FIELDS, _BATCH), lambda f: (0, 0)),
            pl.BlockSpec((_BATCH, 128), lambda f: (f, 0)),
        ],
        out_specs=pl.BlockSpec((1, _EMBED, _BATCH), lambda f: (f, 0, 0)),
    )(idxT, big)


@jax.jit
def _run(idxT, wT):
    w4 = _tc_pack(wT)
    mesh = plsc.VectorSubcoreMesh(core_axis_name="c", subcore_axis_name="s")
    gather = pl.kernel(
        _g_body,
        out_type=jax.ShapeDtypeStruct((_TOTAL, 128), jnp.float32),
        mesh=mesh,
        compiler_params=pltpu.CompilerParams(use_tc_tiling_on_sc=True),
        scratch_types=[
            pltpu.VMEM((_PER_W,), jnp.int32),
            pltpu.VMEM((_PER_W,), jnp.int32),
            pltpu.VMEM((2, 128, 128), jnp.float32),
            pltpu.SemaphoreType.DMA,
        ],
    )
    big = gather(idxT.reshape(-1), w4)
    out3 = _tc_extract(idxT, big)
    return out3.transpose(2, 0, 1)


def kernel(input_tensor, weight):
    idx = input_tensor
    if idx.ndim == 1:
        idx = idx[None, :]
    return _run(idx.T, weight.T)
